# baseline (speedup 1.0000x reference)
import functools
import math

import jax
import jax.numpy as jnp
from jax import lax
from jax.experimental import pallas as pl
from jax.experimental.pallas import tpu as pltpu
from jax.experimental.pallas import tpu_sc as plsc

_NC = 2
_NS = 16
_NW = _NC * _NS
_L = 16

_CHUNK = 2048
_UNROLL = 4

_E = math.exp(-0.5)
_K0 = _E / (1.0 + 2.0 * _E)
_K1 = 1.0 / (1.0 + 2.0 * _E)


def _bf16_round(v):
    u = lax.bitcast_convert_type(v, jnp.int32)
    lsb = (u >> 16) & 1
    r = (u + (32767 + lsb)) & jnp.int32(-65536)
    return lax.bitcast_convert_type(r, jnp.float32)


def _splat_body(baseT, flowT, vals, pb, wv, out, acc, stage, rowtmp, ptmp, wvv):
    X = 256
    NPIX = X * X
    N = vals.shape[0]
    i32 = jnp.int32
    fzero = jnp.zeros((_L,), jnp.float32)

    wid = lax.axis_index("s") * _NC + lax.axis_index("c")
    b = wid

    pltpu.sync_copy(pb.at[pl.ds(b * 32, 32)], ptmp)
    pltpu.sync_copy(wv, wvv)

    p0 = ptmp[pl.ds(0, _L)]
    p1 = ptmp[pl.ds(_L, _L)]

    def bcast(v, lane):
        return v.at[jnp.full((_L,), lane, i32)].get(mode="promise_in_bounds")

    r00 = _bf16_round(bcast(p0, 0))
    r01 = _bf16_round(bcast(p0, 1))
    r02 = _bf16_round(bcast(p0, 2))
    r10 = _bf16_round(bcast(p0, 3))
    r11 = _bf16_round(bcast(p0, 4))
    r12 = _bf16_round(bcast(p0, 5))
    s0 = bcast(p0, 9)
    s1 = bcast(p0, 10)
    col_bias = 128.0 - s0
    row_bias = 128.0 - s1

    def zero_body(i, carry):
        acc[pl.ds(i * _L, _L)] = fzero
        return carry

    lax.fori_loop(0, NPIX // _L, zero_body, 0)

    n_chunks = N // _CHUNK
    fb = b * 3 * N

    def chunk_body(ch, carry):
        off = ch * _CHUNK
        pltpu.sync_copy(baseT.at[pl.ds(off, _CHUNK)],
                        stage.at[pl.ds(0, _CHUNK)])
        pltpu.sync_copy(baseT.at[pl.ds(N + off, _CHUNK)],
                        stage.at[pl.ds(_CHUNK, _CHUNK)])
        pltpu.sync_copy(baseT.at[pl.ds(2 * N + off, _CHUNK)],
                        stage.at[pl.ds(2 * _CHUNK, _CHUNK)])
        pltpu.sync_copy(flowT.at[pl.ds(fb + off, _CHUNK)],
                        stage.at[pl.ds(3 * _CHUNK, _CHUNK)])
        pltpu.sync_copy(flowT.at[pl.ds(fb + N + off, _CHUNK)],
                        stage.at[pl.ds(4 * _CHUNK, _CHUNK)])
        pltpu.sync_copy(flowT.at[pl.ds(fb + 2 * N + off, _CHUNK)],
                        stage.at[pl.ds(5 * _CHUNK, _CHUNK)])
        pltpu.sync_copy(vals.at[pl.ds(off, _CHUNK)],
                        stage.at[pl.ds(6 * _CHUNK, _CHUNK)])

        def group_body(g, carry2):
            for u in range(_UNROLL):
                s = (g * _UNROLL + u) * _L
                vx = stage[pl.ds(s, _L)] + stage[pl.ds(3 * _CHUNK + s, _L)]
                vy = (stage[pl.ds(_CHUNK + s, _L)]
                      + stage[pl.ds(4 * _CHUNK + s, _L)])
                vz = (stage[pl.ds(2 * _CHUNK + s, _L)]
                      + stage[pl.ds(5 * _CHUNK + s, _L)])
                vv = stage[pl.ds(6 * _CHUNK + s, _L)]

                vx = _bf16_round(vx - 128.0)
                vy = _bf16_round(vy - 128.0)
                vz = _bf16_round(vz - 128.0)
                colf = vx * r00 + vy * r01 + vz * r02 + col_bias
                rowf = vx * r10 + vy * r11 + vz * r12 + row_bias

                ci_t = colf.astype(jnp.int32)
                cf = ci_t.astype(jnp.float32)
                ci = jnp.where(cf > colf, ci_t - 1, ci_t)
                cf = jnp.where(cf > colf, cf - 1.0, cf)
                ri_t = rowf.astype(jnp.int32)
                rf = ri_t.astype(jnp.float32)
                ri = jnp.where(rf > rowf, ri_t - 1, ri_t)
                rf = jnp.where(rf > rowf, rf - 1.0, rf)
                fc = colf - cf
                fr = rowf - rf

                vg = vv * (1.0 - fr)
                vf = vv * fr
                a00 = vg * (1.0 - fc)
                a01 = vg * fc
                a10 = vf * (1.0 - fc)
                a11 = vf * fc

                qr0 = ri + X
                qr1 = qr0 + 1
                qc0 = ci + X
                qc1 = qc0 + 1
                u2x = jnp.uint32(2 * X)
                vr0 = qr0.astype(jnp.uint32) < u2x
                vr1 = qr1.astype(jnp.uint32) < u2x
                vc0 = qc0.astype(jnp.uint32) < u2x
                vc1 = qc1.astype(jnp.uint32) < u2x
                r0s = (qr0 & (X - 1)) * X
                r1s = (qr1 & (X - 1)) * X
                c0w = qc0 & (X - 1)
                c1w = qc1 & (X - 1)

                m00 = vr0 & vc0
                m10 = vr1 & vc0
                m11 = vr1 & vc1
                m01 = vr0 & vc1

                plsc.addupdate_scatter(acc, [r0s + c0w], a00, mask=m00)
                plsc.addupdate_scatter(acc, [r1s + c0w], a10, mask=m10)
                plsc.addupdate_scatter(acc, [r1s + c1w], a11, mask=m11)
                plsc.addupdate_scatter(acc, [r0s + c1w], a01, mask=m01)
            return carry2

        lax.fori_loop(0, _CHUNK // (_UNROLL * _L), group_body, 0)
        return carry

    lax.fori_loop(0, n_chunks, chunk_body, 0)

    nstripe = X // _L

    def vblur_body(r, carry):
        new_carry = []
        for i in range(nstripe):
            prev, cur = carry[2 * i], carry[2 * i + 1]
            nxt = acc[pl.ds((r + 1) * X + i * _L, _L)]
            acc[pl.ds(r * X + i * _L, _L)] = _K0 * prev + _K1 * cur + _K0 * nxt
            new_carry.extend((cur, nxt))
        return tuple(new_carry)

    init = []
    for i in range(nstripe):
        init.extend((fzero, acc[pl.ds(i * _L, _L)]))
    carry = lax.fori_loop(0, X - 1, vblur_body, tuple(init))
    for i in range(nstripe):
        prev, cur = carry[2 * i], carry[2 * i + 1]
        acc[pl.ds((X - 1) * X + i * _L, _L)] = _K0 * prev + _K1 * cur

    p0b = ptmp[pl.ds(0, _L)]
    p1b = ptmp[pl.ds(_L, _L)]
    xb = [bcast(p0b, 11 + k) for k in range(5)] + \
         [bcast(p1b, k) for k in range(5)]
    a_vec = fzero
    b_vec = fzero
    for jc in range(4):
        h = wvv[pl.ds(640 + jc * _L, _L)]
        for k in range(10):
            h = h + xb[k] * wvv[pl.ds(k * 64 + jc * _L, _L)]
        h = jnp.maximum(h, 0.0)
        a_vec = a_vec + h * wvv[pl.ds(704 + jc * _L, _L)]
        b_vec = b_vec + h * wvv[pl.ds(768 + jc * _L, _L)]
    lanes = jnp.arange(_L, dtype=i32)
    for k in (8, 4, 2, 1):
        perm = lanes ^ k
        a_vec = a_vec + a_vec.at[perm].get(mode="promise_in_bounds")
        b_vec = b_vec + b_vec.at[perm].get(mode="promise_in_bounds")
    a_s = a_vec + bcast(p1b, 5)
    b_s = b_vec + bcast(p1b, 6)

    ak0 = a_s * _K0
    ak1 = a_s * _K1
    rowtmp[pl.ds(0, _L)] = fzero
    rowtmp[pl.ds(16 + X, _L)] = fzero

    def hblur_body(r, carry):
        rb = r * X
        for i in range(nstripe):
            rowtmp[pl.ds(16 + i * _L, _L)] = acc[pl.ds(rb + i * _L, _L)]
        for i in range(nstripe):
            lft = rowtmp[pl.ds(15 + i * _L, _L)]
            ctr = rowtmp[pl.ds(16 + i * _L, _L)]
            rgt = rowtmp[pl.ds(17 + i * _L, _L)]
            acc[pl.ds(rb + i * _L, _L)] = (ak0 * lft + ak1 * ctr + ak0 * rgt) + b_s
        return carry

    lax.fori_loop(0, X, hblur_body, 0)

    pltpu.sync_copy(acc, out.at[pl.ds(b * NPIX, NPIX)])


def kernel(flow, x, inds, values, xsize, rotations, shifts, ctf, ctf_type,
           W1, b1, W2, b2):
    del xsize, ctf_type
    B, N, _ = flow.shape
    X = ctf.shape[-1]
    LAT = x.shape[-1]
    HID = W1.shape[-1]
    assert B == _NW and X == 256 and LAT == 10 and HID == 64

    f32 = jnp.float32
    baseT = inds[:, ::-1].T.astype(f32).reshape(-1)
    flowT = flow.transpose(0, 2, 1).reshape(-1)
    x2 = jnp.squeeze(x).reshape(B, LAT)
    pb = jnp.concatenate(
        [rotations.reshape(B, 9), shifts, x2,
         jnp.broadcast_to(b2, (B, 2)),
         jnp.zeros((B, 9), f32)], axis=1).reshape(-1)
    wv = jnp.concatenate(
        [W1.reshape(-1), b1, W2.T.reshape(-1)])

    mesh = plsc.VectorSubcoreMesh(core_axis_name="c", subcore_axis_name="s")
    run = pl.kernel(
        _splat_body,
        out_type=jax.ShapeDtypeStruct((B * X * X,), f32),
        mesh=mesh,
        compiler_params=pltpu.CompilerParams(needs_layout_passes=False),
        scratch_types=[
            pltpu.VMEM((X * X,), f32),
            pltpu.VMEM((7 * _CHUNK,), f32),
            pltpu.VMEM((288,), f32),
            pltpu.VMEM((32,), f32),
            pltpu.VMEM((832,), f32),
        ],
    )
    out = run(baseT, flowT, values, pb, wv)
    return out.reshape(B, X, X)


# speedup vs baseline: 1.4160x; 1.4160x over previous
import functools
import math

import jax
import jax.numpy as jnp
from jax import lax
from jax.experimental import pallas as pl
from jax.experimental.pallas import tpu as pltpu
from jax.experimental.pallas import tpu_sc as plsc

_NC = 2
_NS = 16
_NW = _NC * _NS
_L = 16

_CHUNK = 2048
_UNROLL = 4

_E = math.exp(-0.5)
_K0 = _E / (1.0 + 2.0 * _E)
_K1 = 1.0 / (1.0 + 2.0 * _E)


def _bf16_round(v):
    u = lax.bitcast_convert_type(v, jnp.int32)
    lsb = (u >> 16) & 1
    r = (u + (32767 + lsb)) & jnp.int32(-65536)
    return lax.bitcast_convert_type(r, jnp.float32)


def _splat_body(shared, flowc, pb, wv, out, acc, stage, rowtmp, ptmp, wvv,
                dsem):
    X = 256
    NPIX = X * X
    N = shared.shape[0] // 4
    i32 = jnp.int32
    fzero = jnp.zeros((_L,), jnp.float32)

    wid = lax.axis_index("s") * _NC + lax.axis_index("c")
    b = wid

    pltpu.sync_copy(pb.at[pl.ds(b * 32, 32)], ptmp)
    pltpu.sync_copy(wv, wvv)

    p0 = ptmp[pl.ds(0, _L)]
    p1 = ptmp[pl.ds(_L, _L)]

    def bcast(v, lane):
        return v.at[jnp.full((_L,), lane, i32)].get(mode="promise_in_bounds")

    r00 = _bf16_round(bcast(p0, 0))
    r01 = _bf16_round(bcast(p0, 1))
    r02 = _bf16_round(bcast(p0, 2))
    r10 = _bf16_round(bcast(p0, 3))
    r11 = _bf16_round(bcast(p0, 4))
    r12 = _bf16_round(bcast(p0, 5))
    s0 = bcast(p0, 9)
    s1 = bcast(p0, 10)
    col_bias = 128.0 - s0
    row_bias = 128.0 - s1

    def zero_body(i, carry):
        acc[pl.ds(i * _L, _L)] = fzero
        return carry

    lax.fori_loop(0, NPIX // _L, zero_body, 0)

    n_chunks = N // _CHUNK
    C7 = 7 * _CHUNK

    def dma_descs(ch, sb):
        d1 = pltpu.make_async_copy(
            shared.at[pl.ds(ch * 4 * _CHUNK, 4 * _CHUNK)],
            stage.at[pl.ds(sb, 4 * _CHUNK)], dsem)
        d2 = pltpu.make_async_copy(
            flowc.at[pl.ds((b * n_chunks + ch) * 3 * _CHUNK, 3 * _CHUNK)],
            stage.at[pl.ds(sb + 4 * _CHUNK, 3 * _CHUNK)], dsem)
        return d1, d2

    for d in dma_descs(0, 0):
        d.start()

    def chunk_body(ch, carry):
        sb = (ch % 2) * C7
        for d in dma_descs(ch, sb):
            d.wait()

        @pl.when(ch + 1 < n_chunks)
        def _():
            for d in dma_descs(ch + 1, C7 - sb):
                d.start()

        def group_body(g, carry2):
            for u in range(_UNROLL):
                s = sb + (g * _UNROLL + u) * _L
                vx = stage[pl.ds(s, _L)] + stage[pl.ds(4 * _CHUNK + s, _L)]
                vy = (stage[pl.ds(_CHUNK + s, _L)]
                      + stage[pl.ds(5 * _CHUNK + s, _L)])
                vz = (stage[pl.ds(2 * _CHUNK + s, _L)]
                      + stage[pl.ds(6 * _CHUNK + s, _L)])
                vv = stage[pl.ds(3 * _CHUNK + s, _L)]

                vx = _bf16_round(vx - 128.0)
                vy = _bf16_round(vy - 128.0)
                vz = _bf16_round(vz - 128.0)
                colf = vx * r00 + vy * r01 + vz * r02 + col_bias
                rowf = vx * r10 + vy * r11 + vz * r12 + row_bias

                ci_t = colf.astype(jnp.int32)
                cf = ci_t.astype(jnp.float32)
                ci = jnp.where(cf > colf, ci_t - 1, ci_t)
                cf = jnp.where(cf > colf, cf - 1.0, cf)
                ri_t = rowf.astype(jnp.int32)
                rf = ri_t.astype(jnp.float32)
                ri = jnp.where(rf > rowf, ri_t - 1, ri_t)
                rf = jnp.where(rf > rowf, rf - 1.0, rf)
                fc = colf - cf
                fr = rowf - rf

                vg = vv * (1.0 - fr)
                vf = vv * fr
                a00 = vg * (1.0 - fc)
                a01 = vg * fc
                a10 = vf * (1.0 - fc)
                a11 = vf * fc

                qr0 = ri + X
                qr1 = qr0 + 1
                qc0 = ci + X
                qc1 = qc0 + 1
                u2x = jnp.uint32(2 * X)
                vr0 = qr0.astype(jnp.uint32) < u2x
                vr1 = qr1.astype(jnp.uint32) < u2x
                vc0 = qc0.astype(jnp.uint32) < u2x
                vc1 = qc1.astype(jnp.uint32) < u2x
                r0s = (qr0 & (X - 1)) * X
                r1s = (qr1 & (X - 1)) * X
                c0w = qc0 & (X - 1)
                c1w = qc1 & (X - 1)

                m00 = vr0 & vc0
                m10 = vr1 & vc0
                m11 = vr1 & vc1
                m01 = vr0 & vc1

                plsc.addupdate_scatter(acc, [r0s + c0w], a00, mask=m00)
                plsc.addupdate_scatter(acc, [r1s + c0w], a10, mask=m10)
                plsc.addupdate_scatter(acc, [r1s + c1w], a11, mask=m11)
                plsc.addupdate_scatter(acc, [r0s + c1w], a01, mask=m01)
            return carry2

        lax.fori_loop(0, _CHUNK // (_UNROLL * _L), group_body, 0)
        return carry

    lax.fori_loop(0, n_chunks, chunk_body, 0)

    nstripe = X // _L

    def vblur_body(r, carry):
        new_carry = []
        for i in range(nstripe):
            prev, cur = carry[2 * i], carry[2 * i + 1]
            nxt = acc[pl.ds((r + 1) * X + i * _L, _L)]
            acc[pl.ds(r * X + i * _L, _L)] = _K0 * prev + _K1 * cur + _K0 * nxt
            new_carry.extend((cur, nxt))
        return tuple(new_carry)

    init = []
    for i in range(nstripe):
        init.extend((fzero, acc[pl.ds(i * _L, _L)]))
    carry = lax.fori_loop(0, X - 1, vblur_body, tuple(init))
    for i in range(nstripe):
        prev, cur = carry[2 * i], carry[2 * i + 1]
        acc[pl.ds((X - 1) * X + i * _L, _L)] = _K0 * prev + _K1 * cur

    p0b = ptmp[pl.ds(0, _L)]
    p1b = ptmp[pl.ds(_L, _L)]
    xb = [bcast(p0b, 11 + k) for k in range(5)] + \
         [bcast(p1b, k) for k in range(5)]
    a_vec = fzero
    b_vec = fzero
    for jc in range(4):
        h = wvv[pl.ds(640 + jc * _L, _L)]
        for k in range(10):
            h = h + xb[k] * wvv[pl.ds(k * 64 + jc * _L, _L)]
        h = jnp.maximum(h, 0.0)
        a_vec = a_vec + h * wvv[pl.ds(704 + jc * _L, _L)]
        b_vec = b_vec + h * wvv[pl.ds(768 + jc * _L, _L)]
    lanes = jnp.arange(_L, dtype=i32)
    for k in (8, 4, 2, 1):
        perm = lanes ^ k
        a_vec = a_vec + a_vec.at[perm].get(mode="promise_in_bounds")
        b_vec = b_vec + b_vec.at[perm].get(mode="promise_in_bounds")
    a_s = a_vec + bcast(p1b, 5)
    b_s = b_vec + bcast(p1b, 6)

    ak0 = a_s * _K0
    ak1 = a_s * _K1
    rowtmp[pl.ds(0, _L)] = fzero
    rowtmp[pl.ds(16 + X, _L)] = fzero

    def hblur_body(r, carry):
        rb = r * X
        ctrs = []
        for i in range(nstripe):
            ctr = acc[pl.ds(rb + i * _L, _L)]
            rowtmp[pl.ds(16 + i * _L, _L)] = ctr
            ctrs.append(ctr)
        for i in range(nstripe):
            lft = rowtmp[pl.ds(15 + i * _L, _L)]
            rgt = rowtmp[pl.ds(17 + i * _L, _L)]
            acc[pl.ds(rb + i * _L, _L)] = (ak0 * lft + ak1 * ctrs[i]
                                           + ak0 * rgt) + b_s
        return carry

    lax.fori_loop(0, X, hblur_body, 0)

    pltpu.sync_copy(acc, out.at[pl.ds(b * NPIX, NPIX)])


def kernel(flow, x, inds, values, xsize, rotations, shifts, ctf, ctf_type,
           W1, b1, W2, b2):
    del xsize, ctf_type
    B, N, _ = flow.shape
    X = ctf.shape[-1]
    LAT = x.shape[-1]
    HID = W1.shape[-1]
    assert B == _NW and X == 256 and LAT == 10 and HID == 64

    f32 = jnp.float32
    NCH = N // _CHUNK
    baseT = inds[:, ::-1].T.astype(f32)
    shared = jnp.concatenate(
        [baseT.reshape(3, NCH, _CHUNK).transpose(1, 0, 2),
         values.reshape(NCH, 1, _CHUNK)], axis=1).reshape(-1)
    flowc = (flow.transpose(0, 2, 1)
             .reshape(B, 3, NCH, _CHUNK)
             .transpose(0, 2, 1, 3).reshape(-1))
    x2 = jnp.squeeze(x).reshape(B, LAT)
    pb = jnp.concatenate(
        [rotations.reshape(B, 9), shifts, x2,
         jnp.broadcast_to(b2, (B, 2)),
         jnp.zeros((B, 9), f32)], axis=1).reshape(-1)
    wv = jnp.concatenate(
        [W1.reshape(-1), b1, W2.T.reshape(-1)])

    mesh = plsc.VectorSubcoreMesh(core_axis_name="c", subcore_axis_name="s")
    run = pl.kernel(
        _splat_body,
        out_type=jax.ShapeDtypeStruct((B * X * X,), f32),
        mesh=mesh,
        compiler_params=pltpu.CompilerParams(needs_layout_passes=False),
        scratch_types=[
            pltpu.VMEM((X * X,), f32),
            pltpu.VMEM((2 * 7 * _CHUNK,), f32),
            pltpu.VMEM((288,), f32),
            pltpu.VMEM((32,), f32),
            pltpu.VMEM((832,), f32),
            pltpu.SemaphoreType.DMA,
        ],
    )
    out = run(shared, flowc, pb, wv)
    return out.reshape(B, X, X)
